# baseline (speedup 1.0000x reference)
import functools

import jax
import jax.numpy as jnp
from jax import lax
from jax.experimental import pallas as pl
from jax.experimental.pallas import tpu as pltpu
from jax.experimental.pallas import tpu_sc as plsc

N = 10000
E = 160000
NFEAT = 256
NHID = 64
NHEADS = 4
ALPHA = 0.2

NC = 2
NT = 16
L = 16

HALF = 2 * NHID
DENW = 16

EPT = E // NT
CHUNK = 80
NCHUNK = EPT // CHUNK
RCH = 40
NRC = N // RCH
RPT = (NRC + NT - 1) // NT

BLKN = 2000

_SC_PARAMS = pltpu.CompilerParams(
    use_tc_tiling_on_sc=False, needs_layout_passes=False)


def _tc_body(x_ref, w2_ref, wa2_ref, wh_ref, s_ref):
    xb = x_ref[...]
    wh_ref[...] = jnp.dot(xb, w2_ref[0], preferred_element_type=jnp.float32)
    s_ref[...] = jnp.dot(xb, wa2_ref[0], preferred_element_type=jnp.float32)


_tc_call = pl.pallas_call(
    _tc_body,
    grid=(NC, N // BLKN),
    in_specs=[
        pl.BlockSpec((BLKN, NFEAT), lambda h, j: (j, 0)),
        pl.BlockSpec((1, NFEAT, HALF), lambda h, j: (h, 0, 0)),
        pl.BlockSpec((1, NFEAT, 4), lambda h, j: (h, 0, 0)),
    ],
    out_specs=[
        pl.BlockSpec((BLKN, HALF), lambda h, j: (h * (N // BLKN) + j, 0)),
        pl.BlockSpec((BLKN, 4), lambda h, j: (h * (N // BLKN) + j, 0)),
    ],
    out_shape=[
        jax.ShapeDtypeStruct((NC * N, HALF), jnp.float32),
        jax.ShapeDtypeStruct((NC * N, 4), jnp.float32),
    ],
)


_mesh = plsc.VectorSubcoreMesh(core_axis_name="c", subcore_axis_name="s")


@functools.partial(
    pl.kernel,
    out_type=(
        jax.ShapeDtypeStruct((NC * E,), jnp.float32),
        jax.ShapeDtypeStruct((NC * E,), jnp.float32),
    ),
    mesh=_mesh,
    compiler_params=_SC_PARAMS,
    scratch_types=[
        pltpu.VMEM((N, 4), jnp.float32),
        pltpu.VMEM((CHUNK,), jnp.int32),
        pltpu.VMEM((CHUNK,), jnp.int32),
        pltpu.VMEM((CHUNK,), jnp.float32),
        pltpu.VMEM((CHUNK,), jnp.float32),
    ],
)
def _sc_weights(src_hbm, dst_hbm, s_hbm, w0_hbm, w1_hbm,
                s_v, si_v, di_v, w0_v, w1_v):
    c = lax.axis_index("c")
    t = lax.axis_index("s")

    pltpu.sync_copy(s_hbm.at[pl.ds(c * N, N)], s_v)

    def _chunk(i, _):
        eoff = t * EPT + i * CHUNK
        pltpu.sync_copy(src_hbm.at[pl.ds(eoff, CHUNK)], si_v)
        pltpu.sync_copy(dst_hbm.at[pl.ds(eoff, CHUNK)], di_v)
        for k in range(CHUNK // L):
            s16 = si_v[pl.ds(k * L, L)]
            d16 = di_v[pl.ds(k * L, L)]
            col0 = jnp.zeros((L,), jnp.int32)
            z0 = (plsc.load_gather(s_v, [s16, col0])
                  + plsc.load_gather(s_v, [d16, col0 + 2]))
            w0_v[pl.ds(k * L, L)] = jnp.exp(jnp.maximum(z0, ALPHA * z0))
            z1 = (plsc.load_gather(s_v, [s16, col0 + 1])
                  + plsc.load_gather(s_v, [d16, col0 + 3]))
            w1_v[pl.ds(k * L, L)] = jnp.exp(jnp.maximum(z1, ALPHA * z1))
        pltpu.sync_copy(w0_v, w0_hbm.at[pl.ds(c * E + eoff, CHUNK)])
        pltpu.sync_copy(w1_v, w1_hbm.at[pl.ds(c * E + eoff, CHUNK)])
        return 0

    lax.fori_loop(0, NCHUNK, _chunk, 0)


@functools.partial(
    pl.kernel,
    out_type=jax.ShapeDtypeStruct((NC * N, HALF), jnp.float32),
    mesh=_mesh,
    compiler_params=_SC_PARAMS,
    scratch_types=[
        pltpu.VMEM((CHUNK,), jnp.int32),
        pltpu.VMEM((CHUNK,), jnp.int32),
        pltpu.VMEM((CHUNK,), jnp.int32),
        pltpu.VMEM((CHUNK,), jnp.float32),
        pltpu.VMEM((CHUNK,), jnp.float32),
        pltpu.VMEM((CHUNK, HALF), jnp.float32),
        pltpu.VMEM((CHUNK, DENW), jnp.float32),
        pltpu.VMEM((RCH, HALF), jnp.float32),
        pltpu.VMEM((RCH, DENW), jnp.float32),
        pltpu.VMEM_SHARED((N, HALF), jnp.float32),
        pltpu.VMEM_SHARED((N, DENW), jnp.float32),
        pltpu.SemaphoreType.DMA,
    ],
)
def _sc_scatter(src_hbm, dst_hbm, wh_hbm, w0_hbm, w1_hbm, out_hbm,
                si_v, di_v, gi_v, w0_v, w1_v, rows_v, den_v,
                nin_v, dnin_v, acc_num, acc_den, sem):
    c = lax.axis_index("c")
    t = lax.axis_index("s")
    lanes = lax.iota(jnp.int32, L)
    unit0 = (lanes == 0).astype(jnp.float32)
    unit1 = (lanes == 1).astype(jnp.float32)
    zeros16 = jnp.zeros((L,), jnp.float32)

    def _zrow(i, _):
        for k in range(HALF // L):
            nin_v[i, pl.ds(k * L, L)] = zeros16
        dnin_v[i, pl.ds(0, L)] = zeros16
        return 0
    lax.fori_loop(0, RCH, _zrow, 0)

    def _zcopy(q, _):
        g = q * NT + t
        @pl.when(g < NRC)
        def _():
            pltpu.sync_copy(nin_v, acc_num.at[pl.ds(g * RCH, RCH)])
            pltpu.sync_copy(dnin_v, acc_den.at[pl.ds(g * RCH, RCH)])
        return 0
    lax.fori_loop(0, RPT, _zcopy, 0)
    plsc.subcore_barrier()

    cbias = c * N

    def _chunk(i, _):
        eoff = t * EPT + i * CHUNK
        pltpu.sync_copy(src_hbm.at[pl.ds(eoff, CHUNK)], si_v)
        pltpu.sync_copy(dst_hbm.at[pl.ds(eoff, CHUNK)], di_v)
        pltpu.sync_copy(w0_hbm.at[pl.ds(c * E + eoff, CHUNK)], w0_v)
        pltpu.sync_copy(w1_hbm.at[pl.ds(c * E + eoff, CHUNK)], w1_v)
        for k in range(CHUNK // L):
            gi_v[pl.ds(k * L, L)] = si_v[pl.ds(k * L, L)] + cbias
        pltpu.async_copy(wh_hbm.at[gi_v], rows_v, sem).wait()

        def _scale(g, _):
            base = g * 4
            for j4 in range(4):
                j = base + j4
                j16 = jnp.full((L,), j, jnp.int32)
                w0s = plsc.load_gather(w0_v, [j16])
                w1s = plsc.load_gather(w1_v, [j16])
                den_v[j, pl.ds(0, L)] = w0s * unit0 + w1s * unit1
                for k in range(NHID // L):
                    rows_v[j, pl.ds(k * L, L)] = (
                        rows_v[j, pl.ds(k * L, L)] * w0s)
                    rows_v[j, pl.ds(NHID + k * L, L)] = (
                        rows_v[j, pl.ds(NHID + k * L, L)] * w1s)
            return 0
        lax.fori_loop(0, CHUNK // 4, _scale, 0)

        pltpu.sync_copy(rows_v, acc_num.at[di_v], add=True)
        pltpu.sync_copy(den_v, acc_den.at[di_v], add=True)
        return 0

    lax.fori_loop(0, NCHUNK, _chunk, 0)
    plsc.subcore_barrier()

    def _nchunk(q, _):
        g = q * NT + t
        @pl.when(g < NRC)
        def _():
            roff = g * RCH
            pltpu.sync_copy(acc_num.at[pl.ds(roff, RCH)], nin_v)
            pltpu.sync_copy(acc_den.at[pl.ds(roff, RCH)], dnin_v)

            def _nrow(i, _):
                i16 = jnp.full((L,), i, jnp.int32)
                d0 = plsc.load_gather(dnin_v, [i16, jnp.zeros((L,), jnp.int32)])
                d1 = plsc.load_gather(dnin_v, [i16, jnp.ones((L,), jnp.int32)])
                r0 = 1.0 / jnp.maximum(d0, 1e-9)
                r1 = 1.0 / jnp.maximum(d1, 1e-9)
                for k in range(NHID // L):
                    nin_v[i, pl.ds(k * L, L)] = nin_v[i, pl.ds(k * L, L)] * r0
                    nin_v[i, pl.ds(NHID + k * L, L)] = (
                        nin_v[i, pl.ds(NHID + k * L, L)] * r1)
                return 0
            lax.fori_loop(0, RCH, _nrow, 0)
            pltpu.sync_copy(nin_v, out_hbm.at[pl.ds(c * N + roff, RCH)])
        return 0

    lax.fori_loop(0, RPT, _nchunk, 0)


def kernel(x, edge_index, W, a):
    src = edge_index[0]
    dst = edge_index[1]
    Wc = W.transpose(1, 0, 2).reshape(NFEAT, NHEADS * NHID)
    W2 = Wc.reshape(NFEAT, NC, HALF).transpose(1, 0, 2)
    u = jnp.einsum("hfk,hk->hf", W, a[:, :NHID])
    v = jnp.einsum("hfk,hk->hf", W, a[:, NHID:])
    wa = jnp.stack([
        jnp.stack([u[0], u[1], v[0], v[1]], axis=1),
        jnp.stack([u[2], u[3], v[2], v[3]], axis=1),
    ])

    wh2, s2 = _tc_call(x, W2, wa)
    w0, w1 = _sc_weights(src, dst, s2)
    out2 = _sc_scatter(src, dst, wh2, w0, w1)
    return jnp.concatenate([out2[:N], out2[N:]], axis=1)


# speedup vs baseline: 1.1859x; 1.1859x over previous
import functools

import jax
import jax.numpy as jnp
from jax import lax
from jax.experimental import pallas as pl
from jax.experimental.pallas import tpu as pltpu
from jax.experimental.pallas import tpu_sc as plsc

N = 10000
E = 160000
NFEAT = 256
NHID = 64
NHEADS = 4
ALPHA = 0.2

NC = 2
NT = 16
L = 16

HALF = 2 * NHID
DENW = 16

EPT = E // NT
CHUNK = 80
NCHUNK = EPT // CHUNK
BCH = 5
NBATCH = NCHUNK // BCH
RCH = 40
NRC = N // RCH
RPT = (NRC + NT - 1) // NT

BLKN = 2000

_SC_PARAMS = pltpu.CompilerParams(
    use_tc_tiling_on_sc=False, needs_layout_passes=False)


def _tc_body(x_ref, w2_ref, wa2_ref, wh_ref, s_ref):
    xb = x_ref[...]
    wh_ref[...] = jnp.dot(xb, w2_ref[0], preferred_element_type=jnp.float32)
    s_ref[...] = jnp.dot(xb, wa2_ref[0], preferred_element_type=jnp.float32)


_tc_call = pl.pallas_call(
    _tc_body,
    grid=(NC, N // BLKN),
    in_specs=[
        pl.BlockSpec((BLKN, NFEAT), lambda h, j: (j, 0)),
        pl.BlockSpec((1, NFEAT, HALF), lambda h, j: (h, 0, 0)),
        pl.BlockSpec((1, NFEAT, 4), lambda h, j: (h, 0, 0)),
    ],
    out_specs=[
        pl.BlockSpec((BLKN, HALF), lambda h, j: (h * (N // BLKN) + j, 0)),
        pl.BlockSpec((BLKN, 4), lambda h, j: (h * (N // BLKN) + j, 0)),
    ],
    out_shape=[
        jax.ShapeDtypeStruct((NC * N, HALF), jnp.float32),
        jax.ShapeDtypeStruct((NC * N, 4), jnp.float32),
    ],
)


_mesh = plsc.VectorSubcoreMesh(core_axis_name="c", subcore_axis_name="s")


@functools.partial(
    pl.kernel,
    out_type=(
        jax.ShapeDtypeStruct((NC * E,), jnp.float32),
        jax.ShapeDtypeStruct((NC * E,), jnp.float32),
        jax.ShapeDtypeStruct((NC * N, DENW), jnp.float32),
    ),
    mesh=_mesh,
    compiler_params=_SC_PARAMS,
    scratch_types=[
        pltpu.VMEM((N, 4), jnp.float32),
        pltpu.VMEM((CHUNK,), jnp.int32),
        pltpu.VMEM((CHUNK,), jnp.int32),
        pltpu.VMEM((CHUNK,), jnp.float32),
        pltpu.VMEM((CHUNK,), jnp.float32),
        pltpu.VMEM((CHUNK, DENW), jnp.float32),
        pltpu.VMEM((RCH, DENW), jnp.float32),
        pltpu.VMEM_SHARED((N, DENW), jnp.float32),
    ],
)
def _sc_weights(src_hbm, dst_hbm, s_hbm, w0_hbm, w1_hbm, den_hbm,
                s_v, si_v, di_v, w0_v, w1_v, den_v, dout_v, acc_den):
    c = lax.axis_index("c")
    t = lax.axis_index("s")
    lanes = lax.iota(jnp.int32, L)
    unit0 = (lanes == 0).astype(jnp.float32)
    unit1 = (lanes == 1).astype(jnp.float32)
    zeros16 = jnp.zeros((L,), jnp.float32)

    pltpu.sync_copy(s_hbm.at[pl.ds(c * N, N)], s_v)

    def _zrow(i, _):
        dout_v[i, pl.ds(0, L)] = zeros16
        return 0
    lax.fori_loop(0, RCH, _zrow, 0)

    def _zcopy(q, _):
        g = q * NT + t
        @pl.when(g < NRC)
        def _():
            pltpu.sync_copy(dout_v, acc_den.at[pl.ds(g * RCH, RCH)])
        return 0
    lax.fori_loop(0, RPT, _zcopy, 0)
    plsc.subcore_barrier()

    def _chunk(i, _):
        eoff = t * EPT + i * CHUNK
        pltpu.sync_copy(src_hbm.at[pl.ds(eoff, CHUNK)], si_v)
        pltpu.sync_copy(dst_hbm.at[pl.ds(eoff, CHUNK)], di_v)
        for k in range(CHUNK // L):
            s16 = si_v[pl.ds(k * L, L)]
            d16 = di_v[pl.ds(k * L, L)]
            col0 = jnp.zeros((L,), jnp.int32)
            z0 = (plsc.load_gather(s_v, [s16, col0])
                  + plsc.load_gather(s_v, [d16, col0 + 2]))
            w0_v[pl.ds(k * L, L)] = jnp.exp(jnp.maximum(z0, ALPHA * z0))
            z1 = (plsc.load_gather(s_v, [s16, col0 + 1])
                  + plsc.load_gather(s_v, [d16, col0 + 3]))
            w1_v[pl.ds(k * L, L)] = jnp.exp(jnp.maximum(z1, ALPHA * z1))
        pltpu.sync_copy(w0_v, w0_hbm.at[pl.ds(c * E + eoff, CHUNK)])
        pltpu.sync_copy(w1_v, w1_hbm.at[pl.ds(c * E + eoff, CHUNK)])

        def _dens(j, _):
            j16 = jnp.full((L,), j, jnp.int32)
            w0s = plsc.load_gather(w0_v, [j16])
            w1s = plsc.load_gather(w1_v, [j16])
            den_v[j, pl.ds(0, L)] = w0s * unit0 + w1s * unit1
            return 0
        lax.fori_loop(0, CHUNK, _dens, 0)
        pltpu.sync_copy(den_v, acc_den.at[di_v], add=True)
        return 0

    lax.fori_loop(0, NCHUNK, _chunk, 0)
    plsc.subcore_barrier()

    def _dcopy(q, _):
        g = q * NT + t
        @pl.when(g < NRC)
        def _():
            pltpu.sync_copy(acc_den.at[pl.ds(g * RCH, RCH)], dout_v)
            pltpu.sync_copy(dout_v, den_hbm.at[pl.ds(c * N + g * RCH, RCH)])
        return 0
    lax.fori_loop(0, RPT, _dcopy, 0)


@functools.partial(
    pl.kernel,
    out_type=jax.ShapeDtypeStruct((NC * N, HALF), jnp.float32),
    mesh=_mesh,
    compiler_params=_SC_PARAMS,
    scratch_types=[
        pltpu.VMEM((BCH, CHUNK), jnp.int32),
        pltpu.VMEM((BCH, CHUNK), jnp.int32),
        pltpu.VMEM((BCH, CHUNK), jnp.float32),
        pltpu.VMEM((BCH, CHUNK), jnp.float32),
        pltpu.VMEM((CHUNK, HALF), jnp.float32),
        pltpu.VMEM((RCH, HALF), jnp.float32),
        pltpu.VMEM((RCH, DENW), jnp.float32),
        pltpu.VMEM_SHARED((N, HALF), jnp.float32),
        pltpu.SemaphoreType.DMA,
    ],
)
def _sc_scatter(src2_hbm, dst2_hbm, wh_hbm, w02_hbm, w12_hbm, den_hbm, out_hbm,
                gi_v, di_v, w0_v, w1_v, rows_v, nin_v, dnin_v, acc_num, sem):
    c = lax.axis_index("c")
    t = lax.axis_index("s")
    zeros16 = jnp.zeros((L,), jnp.float32)

    def _zrow(i, _):
        for k in range(HALF // L):
            nin_v[i, pl.ds(k * L, L)] = zeros16
        return 0
    lax.fori_loop(0, RCH, _zrow, 0)

    def _zcopy(q, _):
        g = q * NT + t
        @pl.when(g < NRC)
        def _():
            pltpu.sync_copy(nin_v, acc_num.at[pl.ds(g * RCH, RCH)])
        return 0
    lax.fori_loop(0, RPT, _zcopy, 0)
    plsc.subcore_barrier()

    cbias = c * N

    def _batch(b, _):
        crow = t * NCHUNK + b * BCH
        pltpu.sync_copy(src2_hbm.at[pl.ds(crow, BCH)], gi_v)
        pltpu.sync_copy(dst2_hbm.at[pl.ds(crow, BCH)], di_v)
        pltpu.sync_copy(w02_hbm.at[pl.ds(c * (E // CHUNK) + crow, BCH)], w0_v)
        pltpu.sync_copy(w12_hbm.at[pl.ds(c * (E // CHUNK) + crow, BCH)], w1_v)

        def _chunk(j, _):
            def _bias(k, _):
                gi_v[j, pl.ds(k * L, L)] = gi_v[j, pl.ds(k * L, L)] + cbias
                return 0
            lax.fori_loop(0, CHUNK // L, _bias, 0)
            pltpu.async_copy(wh_hbm.at[gi_v.at[j]], rows_v, sem).wait()

            def _scale(e, _):
                e16 = jnp.full((L,), e, jnp.int32)
                w0s = plsc.load_gather(w0_v, [jnp.full((L,), j, jnp.int32), e16])
                w1s = plsc.load_gather(w1_v, [jnp.full((L,), j, jnp.int32), e16])
                for k in range(NHID // L):
                    rows_v[e, pl.ds(k * L, L)] = (
                        rows_v[e, pl.ds(k * L, L)] * w0s)
                    rows_v[e, pl.ds(NHID + k * L, L)] = (
                        rows_v[e, pl.ds(NHID + k * L, L)] * w1s)
                return 0
            lax.fori_loop(0, CHUNK, _scale, 0)

            pltpu.sync_copy(rows_v, acc_num.at[di_v.at[j]], add=True)
            return 0

        lax.fori_loop(0, BCH, _chunk, 0)
        return 0

    lax.fori_loop(0, NBATCH, _batch, 0)
    plsc.subcore_barrier()

    def _nchunk(q, _):
        g = q * NT + t
        @pl.when(g < NRC)
        def _():
            roff = g * RCH
            pltpu.sync_copy(acc_num.at[pl.ds(roff, RCH)], nin_v)
            pltpu.sync_copy(den_hbm.at[pl.ds(c * N + roff, RCH)], dnin_v)

            def _nrow(i, _):
                i16 = jnp.full((L,), i, jnp.int32)
                d0 = plsc.load_gather(dnin_v, [i16, jnp.zeros((L,), jnp.int32)])
                d1 = plsc.load_gather(dnin_v, [i16, jnp.ones((L,), jnp.int32)])
                r0 = 1.0 / jnp.maximum(d0, 1e-9)
                r1 = 1.0 / jnp.maximum(d1, 1e-9)
                for k in range(NHID // L):
                    nin_v[i, pl.ds(k * L, L)] = nin_v[i, pl.ds(k * L, L)] * r0
                    nin_v[i, pl.ds(NHID + k * L, L)] = (
                        nin_v[i, pl.ds(NHID + k * L, L)] * r1)
                return 0
            lax.fori_loop(0, RCH, _nrow, 0)
            pltpu.sync_copy(nin_v, out_hbm.at[pl.ds(c * N + roff, RCH)])
        return 0

    lax.fori_loop(0, RPT, _nchunk, 0)


def kernel(x, edge_index, W, a):
    src = edge_index[0]
    dst = edge_index[1]
    Wc = W.transpose(1, 0, 2).reshape(NFEAT, NHEADS * NHID)
    W2 = Wc.reshape(NFEAT, NC, HALF).transpose(1, 0, 2)
    u = jnp.einsum("hfk,hk->hf", W, a[:, :NHID])
    v = jnp.einsum("hfk,hk->hf", W, a[:, NHID:])
    wa = jnp.stack([
        jnp.stack([u[0], u[1], v[0], v[1]], axis=1),
        jnp.stack([u[2], u[3], v[2], v[3]], axis=1),
    ])

    wh2, s2 = _tc_call(x, W2, wa)
    w0, w1, den = _sc_weights(src, dst, s2)
    out2 = _sc_scatter(
        src.reshape(E // CHUNK, CHUNK),
        dst.reshape(E // CHUNK, CHUNK),
        wh2,
        w0.reshape(NC * E // CHUNK, CHUNK),
        w1.reshape(NC * E // CHUNK, CHUNK),
        den,
    )
    return jnp.concatenate([out2[:N], out2[N:]], axis=1)


# speedup vs baseline: 1.3211x; 1.1140x over previous
import functools

import jax
import jax.numpy as jnp
from jax import lax
from jax.experimental import pallas as pl
from jax.experimental.pallas import tpu as pltpu
from jax.experimental.pallas import tpu_sc as plsc

N = 10000
E = 160000
NFEAT = 256
NHID = 64
NHEADS = 4
ALPHA = 0.2

NC = 2
NT = 16
L = 16

HALF = 2 * NHID
DENW = 16

EPT = E // NT
CHUNK = 80
NCHUNK = EPT // CHUNK

CH2 = 40
NCH2 = EPT // CH2
BCH2 = 10
NB2 = NCH2 // BCH2
PAIRS = BCH2 // 2
RCH = 40
NRC = N // RCH
RPT = (NRC + NT - 1) // NT

BLKN = 2000

_SC_PARAMS = pltpu.CompilerParams(
    use_tc_tiling_on_sc=False, needs_layout_passes=False)


def _tc_body(x_ref, w2_ref, wa2_ref, wh_ref, s_ref):
    xb = x_ref[...]
    wh_ref[...] = jnp.dot(xb, w2_ref[0], preferred_element_type=jnp.float32)
    s_ref[...] = jnp.dot(xb, wa2_ref[0], preferred_element_type=jnp.float32)


_tc_call = pl.pallas_call(
    _tc_body,
    grid=(NC, N // BLKN),
    in_specs=[
        pl.BlockSpec((BLKN, NFEAT), lambda h, j: (j, 0)),
        pl.BlockSpec((1, NFEAT, HALF), lambda h, j: (h, 0, 0)),
        pl.BlockSpec((1, NFEAT, 4), lambda h, j: (h, 0, 0)),
    ],
    out_specs=[
        pl.BlockSpec((BLKN, HALF), lambda h, j: (h * (N // BLKN) + j, 0)),
        pl.BlockSpec((BLKN, 4), lambda h, j: (h * (N // BLKN) + j, 0)),
    ],
    out_shape=[
        jax.ShapeDtypeStruct((NC * N, HALF), jnp.float32),
        jax.ShapeDtypeStruct((NC * N, 4), jnp.float32),
    ],
)


_mesh = plsc.VectorSubcoreMesh(core_axis_name="c", subcore_axis_name="s")


@functools.partial(
    pl.kernel,
    out_type=(
        jax.ShapeDtypeStruct((NC * E,), jnp.float32),
        jax.ShapeDtypeStruct((NC * E,), jnp.float32),
        jax.ShapeDtypeStruct((NC * N, DENW), jnp.float32),
    ),
    mesh=_mesh,
    compiler_params=_SC_PARAMS,
    scratch_types=[
        pltpu.VMEM((N, 4), jnp.float32),
        pltpu.VMEM((CHUNK,), jnp.int32),
        pltpu.VMEM((CHUNK,), jnp.int32),
        pltpu.VMEM((CHUNK,), jnp.float32),
        pltpu.VMEM((CHUNK,), jnp.float32),
        pltpu.VMEM((CHUNK, DENW), jnp.float32),
        pltpu.VMEM((RCH, DENW), jnp.float32),
        pltpu.VMEM_SHARED((N, DENW), jnp.float32),
    ],
)
def _sc_weights(src_hbm, dst_hbm, s_hbm, w0_hbm, w1_hbm, den_hbm,
                s_v, si_v, di_v, w0_v, w1_v, den_v, dout_v, acc_den):
    c = lax.axis_index("c")
    t = lax.axis_index("s")
    lanes = lax.iota(jnp.int32, L)
    unit0 = (lanes == 0).astype(jnp.float32)
    unit1 = (lanes == 1).astype(jnp.float32)
    zeros16 = jnp.zeros((L,), jnp.float32)

    pltpu.sync_copy(s_hbm.at[pl.ds(c * N, N)], s_v)

    def _zrow(i, _):
        dout_v[i, pl.ds(0, L)] = zeros16
        return 0
    lax.fori_loop(0, RCH, _zrow, 0)

    def _zcopy(q, _):
        g = q * NT + t
        @pl.when(g < NRC)
        def _():
            pltpu.sync_copy(dout_v, acc_den.at[pl.ds(g * RCH, RCH)])
        return 0
    lax.fori_loop(0, RPT, _zcopy, 0)
    plsc.subcore_barrier()

    def _chunk(i, _):
        eoff = t * EPT + i * CHUNK
        pltpu.sync_copy(src_hbm.at[pl.ds(eoff, CHUNK)], si_v)
        pltpu.sync_copy(dst_hbm.at[pl.ds(eoff, CHUNK)], di_v)
        for k in range(CHUNK // L):
            s16 = si_v[pl.ds(k * L, L)]
            d16 = di_v[pl.ds(k * L, L)]
            col0 = jnp.zeros((L,), jnp.int32)
            z0 = (plsc.load_gather(s_v, [s16, col0])
                  + plsc.load_gather(s_v, [d16, col0 + 2]))
            w0_v[pl.ds(k * L, L)] = jnp.exp(jnp.maximum(z0, ALPHA * z0))
            z1 = (plsc.load_gather(s_v, [s16, col0 + 1])
                  + plsc.load_gather(s_v, [d16, col0 + 3]))
            w1_v[pl.ds(k * L, L)] = jnp.exp(jnp.maximum(z1, ALPHA * z1))
        pltpu.sync_copy(w0_v, w0_hbm.at[pl.ds(c * E + eoff, CHUNK)])
        pltpu.sync_copy(w1_v, w1_hbm.at[pl.ds(c * E + eoff, CHUNK)])

        def _dens(j, _):
            j16 = jnp.full((L,), j, jnp.int32)
            w0s = plsc.load_gather(w0_v, [j16])
            w1s = plsc.load_gather(w1_v, [j16])
            den_v[j, pl.ds(0, L)] = w0s * unit0 + w1s * unit1
            return 0
        lax.fori_loop(0, CHUNK, _dens, 0)
        pltpu.sync_copy(den_v, acc_den.at[di_v], add=True)
        return 0

    lax.fori_loop(0, NCHUNK, _chunk, 0)
    plsc.subcore_barrier()

    def _dcopy(q, _):
        g = q * NT + t
        @pl.when(g < NRC)
        def _():
            pltpu.sync_copy(acc_den.at[pl.ds(g * RCH, RCH)], dout_v)
            pltpu.sync_copy(dout_v, den_hbm.at[pl.ds(c * N + g * RCH, RCH)])
        return 0
    lax.fori_loop(0, RPT, _dcopy, 0)


@functools.partial(
    pl.kernel,
    out_type=jax.ShapeDtypeStruct((NC * N, HALF), jnp.float32),
    mesh=_mesh,
    compiler_params=_SC_PARAMS,
    scratch_types=[
        pltpu.VMEM((BCH2 * CH2,), jnp.int32),
        pltpu.VMEM((BCH2, CH2), jnp.int32),
        pltpu.VMEM((BCH2 * CH2,), jnp.float32),
        pltpu.VMEM((BCH2 * CH2,), jnp.float32),
        pltpu.VMEM((CH2, HALF), jnp.float32),
        pltpu.VMEM((CH2, HALF), jnp.float32),
        pltpu.VMEM((RCH, HALF), jnp.float32),
        pltpu.VMEM((RCH, DENW), jnp.float32),
        pltpu.VMEM_SHARED((N, HALF), jnp.float32),
        pltpu.SemaphoreType.DMA,
        pltpu.SemaphoreType.DMA,
        pltpu.SemaphoreType.DMA,
        pltpu.SemaphoreType.DMA,
    ],
)
def _sc_scatter(src_hbm, dst2_hbm, wh_hbm, w0_hbm, w1_hbm, den_hbm, out_hbm,
                gi_v, di_v, w0_v, w1_v, rows_a, rows_b, nin_v, dnin_v,
                acc_num, sem_ga, sem_gb, sem_sa, sem_sb):
    c = lax.axis_index("c")
    t = lax.axis_index("s")
    zeros16 = jnp.zeros((L,), jnp.float32)

    def _zrow(i, _):
        for k in range(HALF // L):
            nin_v[i, pl.ds(k * L, L)] = zeros16
        return 0
    lax.fori_loop(0, RCH, _zrow, 0)

    def _zcopy(q, _):
        g = q * NT + t
        @pl.when(g < NRC)
        def _():
            pltpu.sync_copy(nin_v, acc_num.at[pl.ds(g * RCH, RCH)])
        return 0
    lax.fori_loop(0, RPT, _zcopy, 0)
    plsc.subcore_barrier()

    cbias = c * N
    BE = BCH2 * CH2

    def _batch(b, _):
        ebase = t * EPT + b * BE
        rbase = t * NCH2 + b * BCH2

        @pl.when(b > 0)
        def _():
            pltpu.make_async_copy(
                rows_a, acc_num.at[di_v.at[BCH2 - 2]], sem_sa).wait()
            pltpu.make_async_copy(
                rows_b, acc_num.at[di_v.at[BCH2 - 1]], sem_sb).wait()

        pltpu.sync_copy(src_hbm.at[pl.ds(ebase, BE)], gi_v)
        pltpu.sync_copy(dst2_hbm.at[pl.ds(rbase, BCH2)], di_v)
        pltpu.sync_copy(w0_hbm.at[pl.ds(c * E + ebase, BE)], w0_v)
        pltpu.sync_copy(w1_hbm.at[pl.ds(c * E + ebase, BE)], w1_v)
        def _bias(k, _):
            gi_v[pl.ds(k * L, L)] = gi_v[pl.ds(k * L, L)] + cbias
            return 0
        lax.fori_loop(0, BE // L, _bias, 0)

        bufs = ((rows_a, sem_ga, sem_sa), (rows_b, sem_gb, sem_sb))

        def _pair(q, _):
            for par in range(2):
                rbuf, gsem, ssem = bufs[par]
                ch = q * 2 + par

                @pl.when(q > 0)
                def _():
                    pltpu.make_async_copy(
                        rbuf, acc_num.at[di_v.at[ch]], ssem).wait()
                pltpu.async_copy(
                    wh_hbm.at[gi_v.at[pl.ds(ch * CH2, CH2)]], rbuf, gsem)

            for par in range(2):
                rbuf, gsem, ssem = bufs[par]
                ch = q * 2 + par
                pltpu.make_async_copy(
                    wh_hbm.at[gi_v.at[pl.ds(ch * CH2, CH2)]], rbuf, gsem).wait()

                def _scale(e, _):
                    we = ch * CH2 + e
                    w0s = plsc.load_gather(w0_v, [jnp.full((L,), we, jnp.int32)])
                    w1s = plsc.load_gather(w1_v, [jnp.full((L,), we, jnp.int32)])
                    for k in range(NHID // L):
                        rbuf[e, pl.ds(k * L, L)] = (
                            rbuf[e, pl.ds(k * L, L)] * w0s)
                        rbuf[e, pl.ds(NHID + k * L, L)] = (
                            rbuf[e, pl.ds(NHID + k * L, L)] * w1s)
                    return 0
                lax.fori_loop(0, CH2, _scale, 0)

                pltpu.async_copy(
                    rbuf, acc_num.at[di_v.at[ch]], ssem, add=True)
            return 0

        lax.fori_loop(0, PAIRS, _pair, 0)
        return 0

    lax.fori_loop(0, NB2, _batch, 0)

    pltpu.make_async_copy(rows_a, acc_num.at[di_v.at[BCH2 - 2]], sem_sa).wait()
    pltpu.make_async_copy(rows_b, acc_num.at[di_v.at[BCH2 - 1]], sem_sb).wait()
    plsc.subcore_barrier()

    def _nchunk(q, _):
        g = q * NT + t
        @pl.when(g < NRC)
        def _():
            roff = g * RCH
            pltpu.sync_copy(acc_num.at[pl.ds(roff, RCH)], nin_v)
            pltpu.sync_copy(den_hbm.at[pl.ds(c * N + roff, RCH)], dnin_v)

            def _nrow(i, _):
                i16 = jnp.full((L,), i, jnp.int32)
                d0 = plsc.load_gather(dnin_v, [i16, jnp.zeros((L,), jnp.int32)])
                d1 = plsc.load_gather(dnin_v, [i16, jnp.ones((L,), jnp.int32)])
                r0 = 1.0 / jnp.maximum(d0, 1e-9)
                r1 = 1.0 / jnp.maximum(d1, 1e-9)
                for k in range(NHID // L):
                    nin_v[i, pl.ds(k * L, L)] = nin_v[i, pl.ds(k * L, L)] * r0
                    nin_v[i, pl.ds(NHID + k * L, L)] = (
                        nin_v[i, pl.ds(NHID + k * L, L)] * r1)
                return 0
            lax.fori_loop(0, RCH, _nrow, 0)
            pltpu.sync_copy(nin_v, out_hbm.at[pl.ds(c * N + roff, RCH)])
        return 0

    lax.fori_loop(0, RPT, _nchunk, 0)


def kernel(x, edge_index, W, a):
    src = edge_index[0]
    dst = edge_index[1]
    Wc = W.transpose(1, 0, 2).reshape(NFEAT, NHEADS * NHID)
    W2 = Wc.reshape(NFEAT, NC, HALF).transpose(1, 0, 2)
    u = jnp.einsum("hfk,hk->hf", W, a[:, :NHID])
    v = jnp.einsum("hfk,hk->hf", W, a[:, NHID:])
    wa = jnp.stack([
        jnp.stack([u[0], u[1], v[0], v[1]], axis=1),
        jnp.stack([u[2], u[3], v[2], v[3]], axis=1),
    ])

    wh2, s2 = _tc_call(x, W2, wa)
    w0, w1, den = _sc_weights(src, dst, s2)
    out2 = _sc_scatter(src, dst.reshape(E // CH2, CH2), wh2, w0, w1, den)
    return jnp.concatenate([out2[:N], out2[N:]], axis=1)


# speedup vs baseline: 1.8045x; 1.3659x over previous
import functools

import jax
import jax.numpy as jnp
from jax import lax
from jax.experimental import pallas as pl
from jax.experimental.pallas import tpu as pltpu
from jax.experimental.pallas import tpu_sc as plsc

N = 10000
E = 160000
NFEAT = 256
NHID = 64
NHEADS = 4
ALPHA = 0.2

NC = 2
NT = 16
L = 16

HALF = 2 * NHID
DENW = 16

EPT = E // NT
CHUNK = 80
NCHUNK = EPT // CHUNK

WBCH = 5
WBE = WBCH * CHUNK

CH2 = 40
NCH2 = EPT // CH2
BCH2 = 10
NB2 = NCH2 // BCH2
PAIRS = BCH2 // 2
RCH = 40
NRC = N // RCH
RPT = (NRC + NT - 1) // NT

BLKN = 2000

_SC_PARAMS = pltpu.CompilerParams(
    use_tc_tiling_on_sc=False, needs_layout_passes=False)


def _tc_body(x_ref, w2_ref, wa2_ref, wh_ref, s_ref):
    xb = x_ref[...]
    wh_ref[...] = jnp.dot(xb, w2_ref[0], preferred_element_type=jnp.float32)
    s_ref[...] = jnp.dot(xb, wa2_ref[0], preferred_element_type=jnp.float32)


_tc_call = pl.pallas_call(
    _tc_body,
    grid=(NC, N // BLKN),
    in_specs=[
        pl.BlockSpec((BLKN, NFEAT), lambda h, j: (j, 0)),
        pl.BlockSpec((1, NFEAT, HALF), lambda h, j: (h, 0, 0)),
        pl.BlockSpec((1, NFEAT, 4), lambda h, j: (h, 0, 0)),
    ],
    out_specs=[
        pl.BlockSpec((BLKN, HALF), lambda h, j: (h * (N // BLKN) + j, 0)),
        pl.BlockSpec((BLKN, 4), lambda h, j: (h * (N // BLKN) + j, 0)),
    ],
    out_shape=[
        jax.ShapeDtypeStruct((NC * N, HALF), jnp.float32),
        jax.ShapeDtypeStruct((NC * N, 4), jnp.float32),
    ],
)


_mesh = plsc.VectorSubcoreMesh(core_axis_name="c", subcore_axis_name="s")


@functools.partial(
    pl.kernel,
    out_type=(
        jax.ShapeDtypeStruct((NC * E // CHUNK, CHUNK), jnp.float32),
        jax.ShapeDtypeStruct((NC * E // CHUNK, CHUNK), jnp.float32),
        jax.ShapeDtypeStruct((NC * N, DENW), jnp.float32),
    ),
    mesh=_mesh,
    compiler_params=_SC_PARAMS,
    scratch_types=[
        pltpu.VMEM((N, 4), jnp.float32),
        pltpu.VMEM((WBCH, CHUNK), jnp.int32),
        pltpu.VMEM((WBCH, CHUNK), jnp.int32),
        pltpu.VMEM((WBCH, CHUNK), jnp.float32),
        pltpu.VMEM((WBCH, CHUNK), jnp.float32),
        pltpu.VMEM((WBCH, CHUNK, DENW), jnp.float32),
        pltpu.VMEM((RCH, DENW), jnp.float32),
        pltpu.VMEM_SHARED((N, DENW), jnp.float32),
    ],
)
def _sc_weights(src2_hbm, dst2_hbm, s_hbm, w0_hbm, w1_hbm, den_hbm,
                s_v, si_v, di_v, w0_v, w1_v, den3_v, dout_v, acc_den):
    c = lax.axis_index("c")
    t = lax.axis_index("s")
    lanes = lax.iota(jnp.int32, L)
    zeros16 = jnp.zeros((L,), jnp.float32)

    pltpu.sync_copy(s_hbm.at[pl.ds(c * N, N)], s_v)

    def _dz(i, _):
        def _dzc(j, _):
            den3_v[i, j, pl.ds(0, L)] = zeros16
            return 0
        lax.fori_loop(0, CHUNK, _dzc, 0)
        return 0
    lax.fori_loop(0, WBCH, _dz, 0)

    def _zrow(i, _):
        dout_v[i, pl.ds(0, L)] = zeros16
        return 0
    lax.fori_loop(0, RCH, _zrow, 0)

    def _zcopy(q, _):
        g = q * NT + t
        @pl.when(g < NRC)
        def _():
            pltpu.sync_copy(dout_v, acc_den.at[pl.ds(g * RCH, RCH)])
        return 0
    lax.fori_loop(0, RPT, _zcopy, 0)
    plsc.subcore_barrier()

    col16 = jnp.zeros((L,), jnp.int32)

    def _batch(b, _):
        rbase = t * NCHUNK + b * WBCH
        pltpu.sync_copy(src2_hbm.at[pl.ds(rbase, WBCH)], si_v)
        pltpu.sync_copy(dst2_hbm.at[pl.ds(rbase, WBCH)], di_v)

        def _wchunk(j, _):
            j16 = jnp.full((L,), j, jnp.int32)
            for k in range(CHUNK // L):
                s16 = si_v[j, pl.ds(k * L, L)]
                d16 = di_v[j, pl.ds(k * L, L)]
                z0 = (plsc.load_gather(s_v, [s16, col16])
                      + plsc.load_gather(s_v, [d16, col16 + 2]))
                w0g = jnp.exp(jnp.maximum(z0, ALPHA * z0))
                z1 = (plsc.load_gather(s_v, [s16, col16 + 1])
                      + plsc.load_gather(s_v, [d16, col16 + 3]))
                w1g = jnp.exp(jnp.maximum(z1, ALPHA * z1))
                w0_v[j, pl.ds(k * L, L)] = w0g
                w1_v[j, pl.ds(k * L, L)] = w1g
                e16 = lanes + (k * L)
                plsc.store_scatter(den3_v, [j16, e16, col16], w0g)
                plsc.store_scatter(den3_v, [j16, e16, col16 + 1], w1g)
            pltpu.sync_copy(den3_v.at[j], acc_den.at[di_v.at[j]], add=True)
            return 0
        lax.fori_loop(0, WBCH, _wchunk, 0)

        pltpu.sync_copy(w0_v, w0_hbm.at[pl.ds(c * (E // CHUNK) + rbase, WBCH)])
        pltpu.sync_copy(w1_v, w1_hbm.at[pl.ds(c * (E // CHUNK) + rbase, WBCH)])
        return 0

    lax.fori_loop(0, NCHUNK // WBCH, _batch, 0)
    plsc.subcore_barrier()

    def _dcopy(q, _):
        g = q * NT + t
        @pl.when(g < NRC)
        def _():
            pltpu.sync_copy(acc_den.at[pl.ds(g * RCH, RCH)], dout_v)
            pltpu.sync_copy(dout_v, den_hbm.at[pl.ds(c * N + g * RCH, RCH)])
        return 0
    lax.fori_loop(0, RPT, _dcopy, 0)


@functools.partial(
    pl.kernel,
    out_type=jax.ShapeDtypeStruct((NC * N, HALF), jnp.float32),
    mesh=_mesh,
    compiler_params=_SC_PARAMS,
    scratch_types=[
        pltpu.VMEM((BCH2 * CH2,), jnp.int32),
        pltpu.VMEM((BCH2, CH2), jnp.int32),
        pltpu.VMEM((BCH2 * CH2,), jnp.float32),
        pltpu.VMEM((BCH2 * CH2,), jnp.float32),
        pltpu.VMEM((CH2, HALF), jnp.float32),
        pltpu.VMEM((CH2, HALF), jnp.float32),
        pltpu.VMEM((RCH, HALF), jnp.float32),
        pltpu.VMEM((RCH, DENW), jnp.float32),
        pltpu.VMEM_SHARED((N, HALF), jnp.float32),
        pltpu.SemaphoreType.DMA,
        pltpu.SemaphoreType.DMA,
        pltpu.SemaphoreType.DMA,
        pltpu.SemaphoreType.DMA,
    ],
)
def _sc_scatter(src_hbm, dst2_hbm, wh_hbm, w0_hbm, w1_hbm, den_hbm, out_hbm,
                gi_v, di_v, w0_v, w1_v, rows_a, rows_b, nin_v, dnin_v,
                acc_num, sem_ga, sem_gb, sem_sa, sem_sb):
    c = lax.axis_index("c")
    t = lax.axis_index("s")
    zeros16 = jnp.zeros((L,), jnp.float32)

    def _zrow(i, _):
        for k in range(HALF // L):
            nin_v[i, pl.ds(k * L, L)] = zeros16
        return 0
    lax.fori_loop(0, RCH, _zrow, 0)

    def _zcopy(q, _):
        g = q * NT + t
        @pl.when(g < NRC)
        def _():
            pltpu.sync_copy(nin_v, acc_num.at[pl.ds(g * RCH, RCH)])
        return 0
    lax.fori_loop(0, RPT, _zcopy, 0)
    plsc.subcore_barrier()

    cbias = c * N
    BE = BCH2 * CH2

    def _batch(b, _):
        ebase = t * EPT + b * BE
        rbase = t * NCH2 + b * BCH2

        @pl.when(b > 0)
        def _():
            pltpu.make_async_copy(
                rows_a, acc_num.at[di_v.at[BCH2 - 2]], sem_sa).wait()
            pltpu.make_async_copy(
                rows_b, acc_num.at[di_v.at[BCH2 - 1]], sem_sb).wait()

        pltpu.sync_copy(src_hbm.at[pl.ds(ebase, BE)], gi_v)
        pltpu.sync_copy(dst2_hbm.at[pl.ds(rbase, BCH2)], di_v)
        pltpu.sync_copy(w0_hbm.at[pl.ds(c * E + ebase, BE)], w0_v)
        pltpu.sync_copy(w1_hbm.at[pl.ds(c * E + ebase, BE)], w1_v)
        def _bias(k, _):
            gi_v[pl.ds(k * L, L)] = gi_v[pl.ds(k * L, L)] + cbias
            return 0
        lax.fori_loop(0, BE // L, _bias, 0)

        bufs = ((rows_a, sem_ga, sem_sa), (rows_b, sem_gb, sem_sb))

        def _pair(q, _):
            for par in range(2):
                rbuf, gsem, ssem = bufs[par]
                ch = q * 2 + par

                @pl.when(q > 0)
                def _():
                    pltpu.make_async_copy(
                        rbuf, acc_num.at[di_v.at[ch]], ssem).wait()
                pltpu.async_copy(
                    wh_hbm.at[gi_v.at[pl.ds(ch * CH2, CH2)]], rbuf, gsem)

            for par in range(2):
                rbuf, gsem, ssem = bufs[par]
                ch = q * 2 + par
                pltpu.make_async_copy(
                    wh_hbm.at[gi_v.at[pl.ds(ch * CH2, CH2)]], rbuf, gsem).wait()

                def _scale(e, _):
                    we = ch * CH2 + e
                    w0s = plsc.load_gather(w0_v, [jnp.full((L,), we, jnp.int32)])
                    w1s = plsc.load_gather(w1_v, [jnp.full((L,), we, jnp.int32)])
                    for k in range(NHID // L):
                        rbuf[e, pl.ds(k * L, L)] = (
                            rbuf[e, pl.ds(k * L, L)] * w0s)
                        rbuf[e, pl.ds(NHID + k * L, L)] = (
                            rbuf[e, pl.ds(NHID + k * L, L)] * w1s)
                    return 0
                lax.fori_loop(0, CH2, _scale, 0)

                pltpu.async_copy(
                    rbuf, acc_num.at[di_v.at[ch]], ssem, add=True)
            return 0

        lax.fori_loop(0, PAIRS, _pair, 0)
        return 0

    lax.fori_loop(0, NB2, _batch, 0)

    pltpu.make_async_copy(rows_a, acc_num.at[di_v.at[BCH2 - 2]], sem_sa).wait()
    pltpu.make_async_copy(rows_b, acc_num.at[di_v.at[BCH2 - 1]], sem_sb).wait()
    plsc.subcore_barrier()

    def _nchunk(q, _):
        g = q * NT + t
        @pl.when(g < NRC)
        def _():
            roff = g * RCH
            pltpu.sync_copy(acc_num.at[pl.ds(roff, RCH)], nin_v)
            pltpu.sync_copy(den_hbm.at[pl.ds(c * N + roff, RCH)], dnin_v)

            def _nrow(i, _):
                i16 = jnp.full((L,), i, jnp.int32)
                d0 = plsc.load_gather(dnin_v, [i16, jnp.zeros((L,), jnp.int32)])
                d1 = plsc.load_gather(dnin_v, [i16, jnp.ones((L,), jnp.int32)])
                r0 = 1.0 / jnp.maximum(d0, 1e-9)
                r1 = 1.0 / jnp.maximum(d1, 1e-9)
                for k in range(NHID // L):
                    nin_v[i, pl.ds(k * L, L)] = nin_v[i, pl.ds(k * L, L)] * r0
                    nin_v[i, pl.ds(NHID + k * L, L)] = (
                        nin_v[i, pl.ds(NHID + k * L, L)] * r1)
                return 0
            lax.fori_loop(0, RCH, _nrow, 0)
            pltpu.sync_copy(nin_v, out_hbm.at[pl.ds(c * N + roff, RCH)])
        return 0

    lax.fori_loop(0, RPT, _nchunk, 0)


def kernel(x, edge_index, W, a):
    src = edge_index[0]
    dst = edge_index[1]
    Wc = W.transpose(1, 0, 2).reshape(NFEAT, NHEADS * NHID)
    W2 = Wc.reshape(NFEAT, NC, HALF).transpose(1, 0, 2)
    u = jnp.einsum("hfk,hk->hf", W, a[:, :NHID])
    v = jnp.einsum("hfk,hk->hf", W, a[:, NHID:])
    wa = jnp.stack([
        jnp.stack([u[0], u[1], v[0], v[1]], axis=1),
        jnp.stack([u[2], u[3], v[2], v[3]], axis=1),
    ])

    wh2, s2 = _tc_call(x, W2, wa)
    w0, w1, den = _sc_weights(
        src.reshape(E // CHUNK, CHUNK), dst.reshape(E // CHUNK, CHUNK), s2)
    out2 = _sc_scatter(src, dst.reshape(E // CH2, CH2), wh2,
                       w0.reshape(NC * E), w1.reshape(NC * E), den)
    return jnp.concatenate([out2[:N], out2[N:]], axis=1)


# speedup vs baseline: 1.8358x; 1.0174x over previous
import functools

import jax
import jax.numpy as jnp
from jax import lax
from jax.experimental import pallas as pl
from jax.experimental.pallas import tpu as pltpu
from jax.experimental.pallas import tpu_sc as plsc

N = 10000
E = 160000
NFEAT = 256
NHID = 64
NHEADS = 4
ALPHA = 0.2

NC = 2
NT = 16
L = 16

HALF = 2 * NHID
DENW = 16

EPT = E // NT
CHUNK = 80
NCHUNK = EPT // CHUNK

WBCH = 5
WBE = WBCH * CHUNK

CH2 = 40
NCH2 = EPT // CH2
BCH2 = 10
NB2 = NCH2 // BCH2
PAIRS = BCH2 // 2
RCH = 40
NRC = N // RCH
RPT = (NRC + NT - 1) // NT

BLKN = 2000

_SC_PARAMS = pltpu.CompilerParams(
    use_tc_tiling_on_sc=False, needs_layout_passes=False)


def _tc_body(x_ref, w2_ref, wa2_ref, wh_ref, s_ref):
    xb = x_ref[...]
    wh_ref[...] = jnp.dot(xb, w2_ref[0], preferred_element_type=jnp.float32)
    s_ref[...] = jnp.dot(xb, wa2_ref[0], preferred_element_type=jnp.float32)


_tc_call = pl.pallas_call(
    _tc_body,
    grid=(NC, N // BLKN),
    in_specs=[
        pl.BlockSpec((BLKN, NFEAT), lambda h, j: (j, 0)),
        pl.BlockSpec((1, NFEAT, HALF), lambda h, j: (h, 0, 0)),
        pl.BlockSpec((1, NFEAT, 4), lambda h, j: (h, 0, 0)),
    ],
    out_specs=[
        pl.BlockSpec((BLKN, HALF), lambda h, j: (h * (N // BLKN) + j, 0)),
        pl.BlockSpec((BLKN, 4), lambda h, j: (h * (N // BLKN) + j, 0)),
    ],
    out_shape=[
        jax.ShapeDtypeStruct((NC * N, HALF), jnp.float32),
        jax.ShapeDtypeStruct((NC * N, 4), jnp.float32),
    ],
)


_mesh = plsc.VectorSubcoreMesh(core_axis_name="c", subcore_axis_name="s")


@functools.partial(
    pl.kernel,
    out_type=(
        jax.ShapeDtypeStruct((NC * E // CHUNK, CHUNK), jnp.float32),
        jax.ShapeDtypeStruct((NC * E // CHUNK, CHUNK), jnp.float32),
        jax.ShapeDtypeStruct((NC * N, DENW), jnp.float32),
    ),
    mesh=_mesh,
    compiler_params=_SC_PARAMS,
    scratch_types=[
        pltpu.VMEM((N, 4), jnp.float32),
        pltpu.VMEM((WBCH, CHUNK), jnp.int32),
        pltpu.VMEM((WBCH, CHUNK), jnp.int32),
        pltpu.VMEM((WBCH, CHUNK), jnp.float32),
        pltpu.VMEM((WBCH, CHUNK), jnp.float32),
        pltpu.VMEM((WBCH, CHUNK, DENW), jnp.float32),
        pltpu.VMEM((RCH, DENW), jnp.float32),
        pltpu.VMEM_SHARED((N, DENW), jnp.float32),
    ],
)
def _sc_weights(src2_hbm, dst2_hbm, s_hbm, w0_hbm, w1_hbm, den_hbm,
                s_v, si_v, di_v, w0_v, w1_v, den3_v, dout_v, acc_den):
    c = lax.axis_index("c")
    t = lax.axis_index("s")
    lanes = lax.iota(jnp.int32, L)
    zeros16 = jnp.zeros((L,), jnp.float32)

    pltpu.sync_copy(s_hbm.at[pl.ds(c * N, N)], s_v)

    def _dz(i, _):
        def _dzc(j, _):
            den3_v[i, j, pl.ds(0, L)] = zeros16
            return 0
        lax.fori_loop(0, CHUNK, _dzc, 0)
        return 0
    lax.fori_loop(0, WBCH, _dz, 0)

    def _zrow(i, _):
        dout_v[i, pl.ds(0, L)] = zeros16
        return 0
    lax.fori_loop(0, RCH, _zrow, 0)

    def _zcopy(q, _):
        g = q * NT + t
        @pl.when(g < NRC)
        def _():
            pltpu.sync_copy(dout_v, acc_den.at[pl.ds(g * RCH, RCH)])
        return 0
    lax.fori_loop(0, RPT, _zcopy, 0)
    plsc.subcore_barrier()

    col16 = jnp.zeros((L,), jnp.int32)

    def _batch(b, _):
        rbase = t * NCHUNK + b * WBCH
        pltpu.sync_copy(src2_hbm.at[pl.ds(rbase, WBCH)], si_v)
        pltpu.sync_copy(dst2_hbm.at[pl.ds(rbase, WBCH)], di_v)

        def _wchunk(j, _):
            j16 = jnp.full((L,), j, jnp.int32)
            for k in range(CHUNK // L):
                s16 = si_v[j, pl.ds(k * L, L)]
                d16 = di_v[j, pl.ds(k * L, L)]
                z0 = (plsc.load_gather(s_v, [s16, col16])
                      + plsc.load_gather(s_v, [d16, col16 + 2]))
                w0g = jnp.exp(jnp.maximum(z0, ALPHA * z0))
                z1 = (plsc.load_gather(s_v, [s16, col16 + 1])
                      + plsc.load_gather(s_v, [d16, col16 + 3]))
                w1g = jnp.exp(jnp.maximum(z1, ALPHA * z1))
                w0_v[j, pl.ds(k * L, L)] = w0g
                w1_v[j, pl.ds(k * L, L)] = w1g
                e16 = lanes + (k * L)
                plsc.store_scatter(den3_v, [j16, e16, col16], w0g)
                plsc.store_scatter(den3_v, [j16, e16, col16 + 1], w1g)
            pltpu.sync_copy(den3_v.at[j], acc_den.at[di_v.at[j]], add=True)
            return 0
        lax.fori_loop(0, WBCH, _wchunk, 0)

        pltpu.sync_copy(w0_v, w0_hbm.at[pl.ds(c * (E // CHUNK) + rbase, WBCH)])
        pltpu.sync_copy(w1_v, w1_hbm.at[pl.ds(c * (E // CHUNK) + rbase, WBCH)])
        return 0

    lax.fori_loop(0, NCHUNK // WBCH, _batch, 0)
    plsc.subcore_barrier()

    def _dcopy(q, _):
        g = q * NT + t
        @pl.when(g < NRC)
        def _():
            pltpu.sync_copy(acc_den.at[pl.ds(g * RCH, RCH)], dout_v)
            pltpu.sync_copy(dout_v, den_hbm.at[pl.ds(c * N + g * RCH, RCH)])
        return 0
    lax.fori_loop(0, RPT, _dcopy, 0)


@functools.partial(
    pl.kernel,
    out_type=jax.ShapeDtypeStruct((NC * N, HALF), jnp.float32),
    mesh=_mesh,
    compiler_params=_SC_PARAMS,
    scratch_types=[
        pltpu.VMEM((BCH2 * CH2,), jnp.int32),
        pltpu.VMEM((BCH2, CH2), jnp.int32),
        pltpu.VMEM((BCH2 * CH2,), jnp.float32),
        pltpu.VMEM((BCH2 * CH2,), jnp.float32),
        pltpu.VMEM((CH2, HALF), jnp.float32),
        pltpu.VMEM((CH2, HALF), jnp.float32),
        pltpu.VMEM((RCH, HALF), jnp.float32),
        pltpu.VMEM((RCH, DENW), jnp.float32),
        pltpu.VMEM_SHARED((N, HALF), jnp.float32),
        pltpu.SemaphoreType.DMA,
        pltpu.SemaphoreType.DMA,
        pltpu.SemaphoreType.DMA,
        pltpu.SemaphoreType.DMA,
    ],
)
def _sc_scatter(src_hbm, dst2_hbm, wh_hbm, w0_hbm, w1_hbm, den_hbm, out_hbm,
                gi_v, di_v, w0_v, w1_v, rows_a, rows_b, nin_v, dnin_v,
                acc_num, sem_ga, sem_gb, sem_sa, sem_sb):
    c = lax.axis_index("c")
    t = lax.axis_index("s")
    zeros16 = jnp.zeros((L,), jnp.float32)

    def _zrow(i, _):
        for k in range(HALF // L):
            nin_v[i, pl.ds(k * L, L)] = zeros16
        return 0
    lax.fori_loop(0, RCH, _zrow, 0)

    def _zcopy(q, _):
        g = q * NT + t
        @pl.when(g < NRC)
        def _():
            pltpu.sync_copy(nin_v, acc_num.at[pl.ds(g * RCH, RCH)])
        return 0
    lax.fori_loop(0, RPT, _zcopy, 0)
    plsc.subcore_barrier()

    cbias = c * N
    BE = BCH2 * CH2

    def _batch(b, _):
        ebase = t * EPT + b * BE
        rbase = t * NCH2 + b * BCH2

        @pl.when(b > 0)
        def _():
            pltpu.make_async_copy(
                rows_a, acc_num.at[di_v.at[BCH2 - 2]], sem_sa).wait()
            pltpu.make_async_copy(
                rows_b, acc_num.at[di_v.at[BCH2 - 1]], sem_sb).wait()

        pltpu.sync_copy(src_hbm.at[pl.ds(ebase, BE)], gi_v)
        pltpu.sync_copy(dst2_hbm.at[pl.ds(rbase, BCH2)], di_v)
        pltpu.sync_copy(w0_hbm.at[pl.ds(c * E + ebase, BE)], w0_v)
        pltpu.sync_copy(w1_hbm.at[pl.ds(c * E + ebase, BE)], w1_v)
        def _bias(k, _):
            gi_v[pl.ds(k * L, L)] = gi_v[pl.ds(k * L, L)] + cbias
            return 0
        lax.fori_loop(0, BE // L, _bias, 0)

        bufs = ((rows_a, sem_ga, sem_sa), (rows_b, sem_gb, sem_sb))

        def _pair(q, _):
            for par in range(2):
                rbuf, gsem, ssem = bufs[par]
                ch = q * 2 + par

                @pl.when(q > 0)
                def _():
                    pltpu.make_async_copy(
                        rbuf, acc_num.at[di_v.at[ch]], ssem).wait()
                pltpu.async_copy(
                    wh_hbm.at[gi_v.at[pl.ds(ch * CH2, CH2)]], rbuf, gsem)

            for par in range(2):
                rbuf, gsem, ssem = bufs[par]
                ch = q * 2 + par
                pltpu.make_async_copy(
                    wh_hbm.at[gi_v.at[pl.ds(ch * CH2, CH2)]], rbuf, gsem).wait()

                def _scale(g, _):
                    for u in range(2):
                        e = g * 2 + u
                        we = ch * CH2 + e
                        we16 = jnp.full((L,), we, jnp.int32)
                        w0s = plsc.load_gather(w0_v, [we16])
                        w1s = plsc.load_gather(w1_v, [we16])
                        for k in range(NHID // L):
                            rbuf[e, pl.ds(k * L, L)] = (
                                rbuf[e, pl.ds(k * L, L)] * w0s)
                            rbuf[e, pl.ds(NHID + k * L, L)] = (
                                rbuf[e, pl.ds(NHID + k * L, L)] * w1s)
                    return 0
                lax.fori_loop(0, CH2 // 2, _scale, 0)

                pltpu.async_copy(
                    rbuf, acc_num.at[di_v.at[ch]], ssem, add=True)
            return 0

        lax.fori_loop(0, PAIRS, _pair, 0)
        return 0

    lax.fori_loop(0, NB2, _batch, 0)

    pltpu.make_async_copy(rows_a, acc_num.at[di_v.at[BCH2 - 2]], sem_sa).wait()
    pltpu.make_async_copy(rows_b, acc_num.at[di_v.at[BCH2 - 1]], sem_sb).wait()
    plsc.subcore_barrier()

    def _nchunk(q, _):
        g = q * NT + t
        @pl.when(g < NRC)
        def _():
            roff = g * RCH
            pltpu.sync_copy(acc_num.at[pl.ds(roff, RCH)], nin_v)
            pltpu.sync_copy(den_hbm.at[pl.ds(c * N + roff, RCH)], dnin_v)

            def _nrow(i, _):
                i16 = jnp.full((L,), i, jnp.int32)
                d0 = plsc.load_gather(dnin_v, [i16, jnp.zeros((L,), jnp.int32)])
                d1 = plsc.load_gather(dnin_v, [i16, jnp.ones((L,), jnp.int32)])
                r0 = 1.0 / jnp.maximum(d0, 1e-9)
                r1 = 1.0 / jnp.maximum(d1, 1e-9)
                for k in range(NHID // L):
                    nin_v[i, pl.ds(k * L, L)] = nin_v[i, pl.ds(k * L, L)] * r0
                    nin_v[i, pl.ds(NHID + k * L, L)] = (
                        nin_v[i, pl.ds(NHID + k * L, L)] * r1)
                return 0
            lax.fori_loop(0, RCH, _nrow, 0)
            pltpu.sync_copy(nin_v, out_hbm.at[pl.ds(c * N + roff, RCH)])
        return 0

    lax.fori_loop(0, RPT, _nchunk, 0)


def kernel(x, edge_index, W, a):
    src = edge_index[0]
    dst = edge_index[1]
    Wc = W.transpose(1, 0, 2).reshape(NFEAT, NHEADS * NHID)
    W2 = Wc.reshape(NFEAT, NC, HALF).transpose(1, 0, 2)
    u = jnp.einsum("hfk,hk->hf", W, a[:, :NHID])
    v = jnp.einsum("hfk,hk->hf", W, a[:, NHID:])
    wa = jnp.stack([
        jnp.stack([u[0], u[1], v[0], v[1]], axis=1),
        jnp.stack([u[2], u[3], v[2], v[3]], axis=1),
    ])

    wh2, s2 = _tc_call(x, W2, wa)
    w0, w1, den = _sc_weights(
        src.reshape(E // CHUNK, CHUNK), dst.reshape(E // CHUNK, CHUNK), s2)
    out2 = _sc_scatter(src, dst.reshape(E // CH2, CH2), wh2,
                       w0.reshape(NC * E), w1.reshape(NC * E), den)
    return jnp.concatenate([out2[:N], out2[N:]], axis=1)


# speedup vs baseline: 1.8734x; 1.0205x over previous
import functools

import jax
import jax.numpy as jnp
from jax import lax
from jax.experimental import pallas as pl
from jax.experimental.pallas import tpu as pltpu
from jax.experimental.pallas import tpu_sc as plsc

N = 10000
E = 160000
NFEAT = 256
NHID = 64
NHEADS = 4
ALPHA = 0.2

NC = 2
NT = 16
L = 16

HALF = 2 * NHID
DENW = 16

EPT = E // NT
CHUNK = 80
NCHUNK = EPT // CHUNK

WBCH = 5
WBE = WBCH * CHUNK

CH2 = 80
NCH2 = EPT // CH2
BCH2 = 4
NB2 = (NCH2 - 1) // BCH2
PAIRS = BCH2 // 2
RCH = 16
NRC = N // RCH
RPT = (NRC + NT - 1) // NT
WCH = 80
NWC = N // WCH
WPT = (NWC + NT - 1) // NT

BLKN = 2000

_SC_PARAMS = pltpu.CompilerParams(
    use_tc_tiling_on_sc=False, needs_layout_passes=False)


def _tc_body(x_ref, w2_ref, wa2_ref, wh_ref, s_ref):
    xb = x_ref[...]
    wh_ref[...] = jnp.dot(xb, w2_ref[0], preferred_element_type=jnp.float32)
    s_ref[...] = jnp.dot(xb, wa2_ref[0], preferred_element_type=jnp.float32)


_tc_call = pl.pallas_call(
    _tc_body,
    grid=(NC, N // BLKN),
    in_specs=[
        pl.BlockSpec((BLKN, NFEAT), lambda h, j: (j, 0)),
        pl.BlockSpec((1, NFEAT, HALF), lambda h, j: (h, 0, 0)),
        pl.BlockSpec((1, NFEAT, 4), lambda h, j: (h, 0, 0)),
    ],
    out_specs=[
        pl.BlockSpec((BLKN, HALF), lambda h, j: (h * (N // BLKN) + j, 0)),
        pl.BlockSpec((BLKN, 4), lambda h, j: (h * (N // BLKN) + j, 0)),
    ],
    out_shape=[
        jax.ShapeDtypeStruct((NC * N, HALF), jnp.float32),
        jax.ShapeDtypeStruct((NC * N, 4), jnp.float32),
    ],
)


def _norm_body(num_ref, den_ref, out_ref):
    num = num_ref[...]
    den = den_ref[...]
    r0 = 1.0 / jnp.maximum(den[:, 0:1], 1e-9)
    r1 = 1.0 / jnp.maximum(den[:, 1:2], 1e-9)
    out_ref[...] = jnp.concatenate(
        [num[:, :NHID] * r0, num[:, NHID:] * r1], axis=1)


_tc_norm = pl.pallas_call(
    _norm_body,
    grid=(NC * N // BLKN,),
    in_specs=[
        pl.BlockSpec((BLKN, HALF), lambda j: (j, 0)),
        pl.BlockSpec((BLKN, DENW), lambda j: (j, 0)),
    ],
    out_specs=pl.BlockSpec((BLKN, HALF), lambda j: (j, 0)),
    out_shape=jax.ShapeDtypeStruct((NC * N, HALF), jnp.float32),
)


_mesh = plsc.VectorSubcoreMesh(core_axis_name="c", subcore_axis_name="s")


@functools.partial(
    pl.kernel,
    out_type=(
        jax.ShapeDtypeStruct((NC * E // CHUNK, CHUNK), jnp.float32),
        jax.ShapeDtypeStruct((NC * E // CHUNK, CHUNK), jnp.float32),
        jax.ShapeDtypeStruct((NC * N, DENW), jnp.float32),
    ),
    mesh=_mesh,
    compiler_params=_SC_PARAMS,
    scratch_types=[
        pltpu.VMEM((N, 4), jnp.float32),
        pltpu.VMEM((WBCH, CHUNK), jnp.int32),
        pltpu.VMEM((WBCH, CHUNK), jnp.int32),
        pltpu.VMEM((WBCH, CHUNK), jnp.float32),
        pltpu.VMEM((WBCH, CHUNK), jnp.float32),
        pltpu.VMEM((WBCH, CHUNK, DENW), jnp.float32),
        pltpu.VMEM((RCH, DENW), jnp.float32),
        pltpu.VMEM_SHARED((N, DENW), jnp.float32),
    ],
)
def _sc_weights(src2_hbm, dst2_hbm, s_hbm, w0_hbm, w1_hbm, den_hbm,
                s_v, si_v, di_v, w0_v, w1_v, den3_v, dout_v, acc_den):
    c = lax.axis_index("c")
    t = lax.axis_index("s")
    lanes = lax.iota(jnp.int32, L)
    zeros16 = jnp.zeros((L,), jnp.float32)

    pltpu.sync_copy(s_hbm.at[pl.ds(c * N, N)], s_v)

    def _dz(i, _):
        def _dzc(j, _):
            den3_v[i, j, pl.ds(0, L)] = zeros16
            return 0
        lax.fori_loop(0, CHUNK, _dzc, 0)
        return 0
    lax.fori_loop(0, WBCH, _dz, 0)

    def _zrow(i, _):
        dout_v[i, pl.ds(0, L)] = zeros16
        return 0
    lax.fori_loop(0, RCH, _zrow, 0)

    def _zcopy(q, _):
        g = q * NT + t
        @pl.when(g < NRC)
        def _():
            pltpu.sync_copy(dout_v, acc_den.at[pl.ds(g * RCH, RCH)])
        return 0
    lax.fori_loop(0, RPT, _zcopy, 0)
    plsc.subcore_barrier()

    col16 = jnp.zeros((L,), jnp.int32)

    def _batch(b, _):
        rbase = t * NCHUNK + b * WBCH
        pltpu.sync_copy(src2_hbm.at[pl.ds(rbase, WBCH)], si_v)
        pltpu.sync_copy(dst2_hbm.at[pl.ds(rbase, WBCH)], di_v)

        def _wchunk(j, _):
            j16 = jnp.full((L,), j, jnp.int32)
            for k in range(CHUNK // L):
                s16 = si_v[j, pl.ds(k * L, L)]
                d16 = di_v[j, pl.ds(k * L, L)]
                z0 = (plsc.load_gather(s_v, [s16, col16])
                      + plsc.load_gather(s_v, [d16, col16 + 2]))
                w0g = jnp.exp(jnp.maximum(z0, ALPHA * z0))
                z1 = (plsc.load_gather(s_v, [s16, col16 + 1])
                      + plsc.load_gather(s_v, [d16, col16 + 3]))
                w1g = jnp.exp(jnp.maximum(z1, ALPHA * z1))
                w0_v[j, pl.ds(k * L, L)] = w0g
                w1_v[j, pl.ds(k * L, L)] = w1g
                e16 = lanes + (k * L)
                plsc.store_scatter(den3_v, [j16, e16, col16], w0g)
                plsc.store_scatter(den3_v, [j16, e16, col16 + 1], w1g)
            pltpu.sync_copy(den3_v.at[j], acc_den.at[di_v.at[j]], add=True)
            return 0
        lax.fori_loop(0, WBCH, _wchunk, 0)

        pltpu.sync_copy(w0_v, w0_hbm.at[pl.ds(c * (E // CHUNK) + rbase, WBCH)])
        pltpu.sync_copy(w1_v, w1_hbm.at[pl.ds(c * (E // CHUNK) + rbase, WBCH)])
        return 0

    lax.fori_loop(0, NCHUNK // WBCH, _batch, 0)
    plsc.subcore_barrier()

    def _dcopy(q, _):
        g = q * NT + t
        @pl.when(g < NWC)
        def _():
            pltpu.sync_copy(acc_den.at[pl.ds(g * WCH, WCH)],
                            den_hbm.at[pl.ds(c * N + g * WCH, WCH)])
        return 0
    lax.fori_loop(0, WPT, _dcopy, 0)


@functools.partial(
    pl.kernel,
    out_type=jax.ShapeDtypeStruct((NC * N, HALF), jnp.float32),
    mesh=_mesh,
    compiler_params=_SC_PARAMS,
    scratch_types=[
        pltpu.VMEM((BCH2 * CH2,), jnp.int32),
        pltpu.VMEM((BCH2, CH2), jnp.int32),
        pltpu.VMEM((BCH2 * CH2,), jnp.float32),
        pltpu.VMEM((BCH2 * CH2,), jnp.float32),
        pltpu.VMEM((CH2, HALF), jnp.float32),
        pltpu.VMEM((CH2, HALF), jnp.float32),
        pltpu.VMEM((RCH, HALF), jnp.float32),
        pltpu.VMEM_SHARED((N, HALF), jnp.float32),
        pltpu.SemaphoreType.DMA,
        pltpu.SemaphoreType.DMA,
        pltpu.SemaphoreType.DMA,
        pltpu.SemaphoreType.DMA,
    ],
)
def _sc_scatter(src_hbm, dst2_hbm, wh_hbm, w0_hbm, w1_hbm, out_hbm,
                gi_v, di_v, w0_v, w1_v, rows_a, rows_b, zin_v,
                acc_num, sem_ga, sem_gb, sem_sa, sem_sb):
    c = lax.axis_index("c")
    t = lax.axis_index("s")
    zeros16 = jnp.zeros((L,), jnp.float32)

    def _zrow(i, _):
        for k in range(HALF // L):
            zin_v[i, pl.ds(k * L, L)] = zeros16
        return 0
    lax.fori_loop(0, RCH, _zrow, 0)

    def _zcopy(q, _):
        g = q * NT + t
        @pl.when(g < NRC)
        def _():
            pltpu.sync_copy(zin_v, acc_num.at[pl.ds(g * RCH, RCH)])
        return 0
    lax.fori_loop(0, RPT, _zcopy, 0)
    plsc.subcore_barrier()

    cbias = c * N
    BE = BCH2 * CH2

    def _batch(b, _):
        ebase = t * EPT + b * BE
        rbase = t * NCH2 + b * BCH2

        @pl.when(b > 0)
        def _():
            pltpu.make_async_copy(
                rows_a, acc_num.at[di_v.at[BCH2 - 2]], sem_sa).wait()
            pltpu.make_async_copy(
                rows_b, acc_num.at[di_v.at[BCH2 - 1]], sem_sb).wait()

        pltpu.sync_copy(src_hbm.at[pl.ds(ebase, BE)], gi_v)
        pltpu.sync_copy(dst2_hbm.at[pl.ds(rbase, BCH2)], di_v)
        pltpu.sync_copy(w0_hbm.at[pl.ds(c * E + ebase, BE)], w0_v)
        pltpu.sync_copy(w1_hbm.at[pl.ds(c * E + ebase, BE)], w1_v)
        def _bias(k, _):
            gi_v[pl.ds(k * L, L)] = gi_v[pl.ds(k * L, L)] + cbias
            return 0
        lax.fori_loop(0, BE // L, _bias, 0)

        bufs = ((rows_a, sem_ga, sem_sa), (rows_b, sem_gb, sem_sb))

        def _pair(q, _):
            for par in range(2):
                rbuf, gsem, ssem = bufs[par]
                ch = q * 2 + par

                @pl.when(q > 0)
                def _():
                    pltpu.make_async_copy(
                        rbuf, acc_num.at[di_v.at[ch]], ssem).wait()
                pltpu.async_copy(
                    wh_hbm.at[gi_v.at[pl.ds(ch * CH2, CH2)]], rbuf, gsem)

            for par in range(2):
                rbuf, gsem, ssem = bufs[par]
                ch = q * 2 + par
                pltpu.make_async_copy(
                    wh_hbm.at[gi_v.at[pl.ds(ch * CH2, CH2)]], rbuf, gsem).wait()

                def _scale(g, _):
                    for u in range(2):
                        e = g * 2 + u
                        we = ch * CH2 + e
                        we16 = jnp.full((L,), we, jnp.int32)
                        w0s = plsc.load_gather(w0_v, [we16])
                        w1s = plsc.load_gather(w1_v, [we16])
                        for k in range(NHID // L):
                            rbuf[e, pl.ds(k * L, L)] = (
                                rbuf[e, pl.ds(k * L, L)] * w0s)
                            rbuf[e, pl.ds(NHID + k * L, L)] = (
                                rbuf[e, pl.ds(NHID + k * L, L)] * w1s)
                    return 0
                lax.fori_loop(0, CH2 // 2, _scale, 0)

                pltpu.async_copy(
                    rbuf, acc_num.at[di_v.at[ch]], ssem, add=True)
            return 0

        lax.fori_loop(0, PAIRS, _pair, 0)
        return 0

    lax.fori_loop(0, NB2, _batch, 0)

    pltpu.make_async_copy(rows_a, acc_num.at[di_v.at[BCH2 - 2]], sem_sa).wait()
    pltpu.make_async_copy(rows_b, acc_num.at[di_v.at[BCH2 - 1]], sem_sb).wait()

    eoff = t * EPT + NB2 * BCH2 * CH2
    pltpu.sync_copy(src_hbm.at[pl.ds(eoff, CH2)], gi_v.at[pl.ds(0, CH2)])
    pltpu.sync_copy(dst2_hbm.at[pl.ds(eoff // CH2, 1)], di_v.at[pl.ds(0, 1)])
    pltpu.sync_copy(w0_hbm.at[pl.ds(c * E + eoff, CH2)], w0_v.at[pl.ds(0, CH2)])
    pltpu.sync_copy(w1_hbm.at[pl.ds(c * E + eoff, CH2)], w1_v.at[pl.ds(0, CH2)])

    def _ebias(k, _):
        gi_v[pl.ds(k * L, L)] = gi_v[pl.ds(k * L, L)] + cbias
        return 0
    lax.fori_loop(0, CH2 // L, _ebias, 0)
    pltpu.async_copy(wh_hbm.at[gi_v.at[pl.ds(0, CH2)]], rows_a, sem_ga).wait()

    def _escale(g, _):
        for u in range(2):
            e = g * 2 + u
            we16 = jnp.full((L,), e, jnp.int32)
            w0s = plsc.load_gather(w0_v, [we16])
            w1s = plsc.load_gather(w1_v, [we16])
            for k in range(NHID // L):
                rows_a[e, pl.ds(k * L, L)] = rows_a[e, pl.ds(k * L, L)] * w0s
                rows_a[e, pl.ds(NHID + k * L, L)] = (
                    rows_a[e, pl.ds(NHID + k * L, L)] * w1s)
        return 0
    lax.fori_loop(0, CH2 // 2, _escale, 0)
    pltpu.sync_copy(rows_a, acc_num.at[di_v.at[0]], add=True)

    plsc.subcore_barrier()

    def _ochunk(q, _):
        g = q * NT + t
        @pl.when(g < NWC)
        def _():
            roff = g * WCH
            pltpu.sync_copy(acc_num.at[pl.ds(roff, WCH)],
                            out_hbm.at[pl.ds(c * N + roff, WCH)])
        return 0
    lax.fori_loop(0, WPT, _ochunk, 0)


def kernel(x, edge_index, W, a):
    src = edge_index[0]
    dst = edge_index[1]
    Wc = W.transpose(1, 0, 2).reshape(NFEAT, NHEADS * NHID)
    W2 = Wc.reshape(NFEAT, NC, HALF).transpose(1, 0, 2)
    u = jnp.einsum("hfk,hk->hf", W, a[:, :NHID])
    v = jnp.einsum("hfk,hk->hf", W, a[:, NHID:])
    wa = jnp.stack([
        jnp.stack([u[0], u[1], v[0], v[1]], axis=1),
        jnp.stack([u[2], u[3], v[2], v[3]], axis=1),
    ])

    wh2, s2 = _tc_call(x, W2, wa)
    w0, w1, den = _sc_weights(
        src.reshape(E // CHUNK, CHUNK), dst.reshape(E // CHUNK, CHUNK), s2)
    num2 = _sc_scatter(src, dst.reshape(E // CH2, CH2), wh2,
                       w0.reshape(NC * E), w1.reshape(NC * E))
    out2 = _tc_norm(num2, den)
    return jnp.concatenate([out2[:N], out2[N:]], axis=1)


# speedup vs baseline: 1.9073x; 1.0181x over previous
import functools

import jax
import jax.numpy as jnp
from jax import lax
from jax.experimental import pallas as pl
from jax.experimental.pallas import tpu as pltpu
from jax.experimental.pallas import tpu_sc as plsc

N = 10000
E = 160000
NFEAT = 256
NHID = 64
NHEADS = 4
ALPHA = 0.2

NC = 2
NT = 16
L = 16

HALF = 2 * NHID
DENW = 16

EPT = E // NT
CHUNK = 80
NCHUNK = EPT // CHUNK

WBCH = 5
WBE = WBCH * CHUNK

CH2 = 80
NCH2 = EPT // CH2
BCH2 = 4
NB2 = (NCH2 - 1) // BCH2
PAIRS = BCH2 // 2
RCH = 16
NRC = N // RCH
RPT = (NRC + NT - 1) // NT
WCH = 80
NWC = N // WCH
WPT = (NWC + NT - 1) // NT

BLKN = 2000

_SC_PARAMS = pltpu.CompilerParams(
    use_tc_tiling_on_sc=False, needs_layout_passes=False)


def _tc_s_body(x_ref, wa2_ref, s_ref):
    s_ref[...] = jnp.dot(x_ref[...], wa2_ref[0],
                         preferred_element_type=jnp.float32)


_tc_s = pl.pallas_call(
    _tc_s_body,
    grid=(NC, N // BLKN),
    in_specs=[
        pl.BlockSpec((BLKN, NFEAT), lambda h, j: (j, 0)),
        pl.BlockSpec((1, NFEAT, 4), lambda h, j: (h, 0, 0)),
    ],
    out_specs=pl.BlockSpec((BLKN, 4), lambda h, j: (h * (N // BLKN) + j, 0)),
    out_shape=jax.ShapeDtypeStruct((NC * N, 4), jnp.float32),
)


def _tc_wh_body(x_ref, w2_ref, wh_ref):
    wh_ref[...] = jnp.dot(x_ref[...], w2_ref[0],
                          preferred_element_type=jnp.float32)


_tc_wh = pl.pallas_call(
    _tc_wh_body,
    grid=(NC, N // BLKN),
    in_specs=[
        pl.BlockSpec((BLKN, NFEAT), lambda h, j: (j, 0)),
        pl.BlockSpec((1, NFEAT, HALF), lambda h, j: (h, 0, 0)),
    ],
    out_specs=pl.BlockSpec((BLKN, HALF), lambda h, j: (h * (N // BLKN) + j, 0)),
    out_shape=jax.ShapeDtypeStruct((NC * N, HALF), jnp.float32),
)


def _norm_body(num_ref, den_ref, out_ref):
    num = num_ref[...]
    den = den_ref[...]
    r0 = 1.0 / jnp.maximum(den[:, 0:1], 1e-9)
    r1 = 1.0 / jnp.maximum(den[:, 1:2], 1e-9)
    out_ref[...] = jnp.concatenate(
        [num[:, :NHID] * r0, num[:, NHID:] * r1], axis=1)


_tc_norm = pl.pallas_call(
    _norm_body,
    grid=(NC, N // BLKN),
    in_specs=[
        pl.BlockSpec((BLKN, HALF), lambda h, j: (h * (N // BLKN) + j, 0)),
        pl.BlockSpec((BLKN, DENW), lambda h, j: (h * (N // BLKN) + j, 0)),
    ],
    out_specs=pl.BlockSpec((BLKN, HALF), lambda h, j: (j, h)),
    out_shape=jax.ShapeDtypeStruct((N, NC * HALF), jnp.float32),
)


_mesh = plsc.VectorSubcoreMesh(core_axis_name="c", subcore_axis_name="s")


@functools.partial(
    pl.kernel,
    out_type=(
        jax.ShapeDtypeStruct((NC * E // CHUNK, CHUNK), jnp.float32),
        jax.ShapeDtypeStruct((NC * E // CHUNK, CHUNK), jnp.float32),
        jax.ShapeDtypeStruct((NC * N, DENW), jnp.float32),
    ),
    mesh=_mesh,
    compiler_params=_SC_PARAMS,
    scratch_types=[
        pltpu.VMEM((N, 4), jnp.float32),
        pltpu.VMEM((WBCH, CHUNK), jnp.int32),
        pltpu.VMEM((WBCH, CHUNK), jnp.int32),
        pltpu.VMEM((WBCH, CHUNK), jnp.float32),
        pltpu.VMEM((WBCH, CHUNK), jnp.float32),
        pltpu.VMEM((WBCH, CHUNK, DENW), jnp.float32),
        pltpu.VMEM((RCH, DENW), jnp.float32),
        pltpu.VMEM_SHARED((N, DENW), jnp.float32),
    ],
)
def _sc_weights(src2_hbm, dst2_hbm, s_hbm, w0_hbm, w1_hbm, den_hbm,
                s_v, si_v, di_v, w0_v, w1_v, den3_v, dout_v, acc_den):
    c = lax.axis_index("c")
    t = lax.axis_index("s")
    lanes = lax.iota(jnp.int32, L)
    zeros16 = jnp.zeros((L,), jnp.float32)

    pltpu.sync_copy(s_hbm.at[pl.ds(c * N, N)], s_v)

    def _dz(i, _):
        def _dzc(j, _):
            den3_v[i, j, pl.ds(0, L)] = zeros16
            return 0
        lax.fori_loop(0, CHUNK, _dzc, 0)
        return 0
    lax.fori_loop(0, WBCH, _dz, 0)

    def _zrow(i, _):
        dout_v[i, pl.ds(0, L)] = zeros16
        return 0
    lax.fori_loop(0, RCH, _zrow, 0)

    def _zcopy(q, _):
        g = q * NT + t
        @pl.when(g < NRC)
        def _():
            pltpu.sync_copy(dout_v, acc_den.at[pl.ds(g * RCH, RCH)])
        return 0
    lax.fori_loop(0, RPT, _zcopy, 0)
    plsc.subcore_barrier()

    col16 = jnp.zeros((L,), jnp.int32)

    def _batch(b, _):
        rbase = t * NCHUNK + b * WBCH
        pltpu.sync_copy(src2_hbm.at[pl.ds(rbase, WBCH)], si_v)
        pltpu.sync_copy(dst2_hbm.at[pl.ds(rbase, WBCH)], di_v)

        def _wchunk(j, _):
            j16 = jnp.full((L,), j, jnp.int32)
            for k in range(CHUNK // L):
                s16 = si_v[j, pl.ds(k * L, L)]
                d16 = di_v[j, pl.ds(k * L, L)]
                z0 = (plsc.load_gather(s_v, [s16, col16])
                      + plsc.load_gather(s_v, [d16, col16 + 2]))
                w0g = jnp.exp(jnp.maximum(z0, ALPHA * z0))
                z1 = (plsc.load_gather(s_v, [s16, col16 + 1])
                      + plsc.load_gather(s_v, [d16, col16 + 3]))
                w1g = jnp.exp(jnp.maximum(z1, ALPHA * z1))
                w0_v[j, pl.ds(k * L, L)] = w0g
                w1_v[j, pl.ds(k * L, L)] = w1g
                e16 = lanes + (k * L)
                plsc.store_scatter(den3_v, [j16, e16, col16], w0g)
                plsc.store_scatter(den3_v, [j16, e16, col16 + 1], w1g)
            pltpu.sync_copy(den3_v.at[j], acc_den.at[di_v.at[j]], add=True)
            return 0
        lax.fori_loop(0, WBCH, _wchunk, 0)

        pltpu.sync_copy(w0_v, w0_hbm.at[pl.ds(c * (E // CHUNK) + rbase, WBCH)])
        pltpu.sync_copy(w1_v, w1_hbm.at[pl.ds(c * (E // CHUNK) + rbase, WBCH)])
        return 0

    lax.fori_loop(0, NCHUNK // WBCH, _batch, 0)
    plsc.subcore_barrier()

    def _dcopy(q, _):
        g = q * NT + t
        @pl.when(g < NWC)
        def _():
            pltpu.sync_copy(acc_den.at[pl.ds(g * WCH, WCH)],
                            den_hbm.at[pl.ds(c * N + g * WCH, WCH)])
        return 0
    lax.fori_loop(0, WPT, _dcopy, 0)


@functools.partial(
    pl.kernel,
    out_type=jax.ShapeDtypeStruct((NC * N, HALF), jnp.float32),
    mesh=_mesh,
    compiler_params=_SC_PARAMS,
    scratch_types=[
        pltpu.VMEM((BCH2 * CH2,), jnp.int32),
        pltpu.VMEM((BCH2, CH2), jnp.int32),
        pltpu.VMEM((BCH2 * CH2,), jnp.float32),
        pltpu.VMEM((BCH2 * CH2,), jnp.float32),
        pltpu.VMEM((CH2, HALF), jnp.float32),
        pltpu.VMEM((CH2, HALF), jnp.float32),
        pltpu.VMEM((RCH, HALF), jnp.float32),
        pltpu.VMEM_SHARED((N, HALF), jnp.float32),
        pltpu.SemaphoreType.DMA,
        pltpu.SemaphoreType.DMA,
        pltpu.SemaphoreType.DMA,
        pltpu.SemaphoreType.DMA,
    ],
)
def _sc_scatter(src_hbm, dst2_hbm, wh_hbm, w0_hbm, w1_hbm, out_hbm,
                gi_v, di_v, w0_v, w1_v, rows_a, rows_b, zin_v,
                acc_num, sem_ga, sem_gb, sem_sa, sem_sb):
    c = lax.axis_index("c")
    t = lax.axis_index("s")
    zeros16 = jnp.zeros((L,), jnp.float32)

    def _zrow(i, _):
        for k in range(HALF // L):
            zin_v[i, pl.ds(k * L, L)] = zeros16
        return 0
    lax.fori_loop(0, RCH, _zrow, 0)

    def _zcopy(q, _):
        g = q * NT + t
        @pl.when(g < NRC)
        def _():
            pltpu.sync_copy(zin_v, acc_num.at[pl.ds(g * RCH, RCH)])
        return 0
    lax.fori_loop(0, RPT, _zcopy, 0)
    plsc.subcore_barrier()

    cbias = c * N
    BE = BCH2 * CH2

    def _batch(b, _):
        ebase = t * EPT + b * BE
        rbase = t * NCH2 + b * BCH2

        @pl.when(b > 0)
        def _():
            pltpu.make_async_copy(
                rows_a, acc_num.at[di_v.at[BCH2 - 2]], sem_sa).wait()
            pltpu.make_async_copy(
                rows_b, acc_num.at[di_v.at[BCH2 - 1]], sem_sb).wait()

        pltpu.sync_copy(src_hbm.at[pl.ds(ebase, BE)], gi_v)
        pltpu.sync_copy(dst2_hbm.at[pl.ds(rbase, BCH2)], di_v)
        pltpu.sync_copy(w0_hbm.at[pl.ds(c * E + ebase, BE)], w0_v)
        pltpu.sync_copy(w1_hbm.at[pl.ds(c * E + ebase, BE)], w1_v)
        def _bias(k, _):
            gi_v[pl.ds(k * L, L)] = gi_v[pl.ds(k * L, L)] + cbias
            return 0
        lax.fori_loop(0, BE // L, _bias, 0)

        bufs = ((rows_a, sem_ga, sem_sa), (rows_b, sem_gb, sem_sb))

        def _pair(q, _):
            for par in range(2):
                rbuf, gsem, ssem = bufs[par]
                ch = q * 2 + par

                @pl.when(q > 0)
                def _():
                    pltpu.make_async_copy(
                        rbuf, acc_num.at[di_v.at[ch]], ssem).wait()
                pltpu.async_copy(
                    wh_hbm.at[gi_v.at[pl.ds(ch * CH2, CH2)]], rbuf, gsem)

            for par in range(2):
                rbuf, gsem, ssem = bufs[par]
                ch = q * 2 + par
                pltpu.make_async_copy(
                    wh_hbm.at[gi_v.at[pl.ds(ch * CH2, CH2)]], rbuf, gsem).wait()

                def _scale(g, _):
                    for u in range(2):
                        e = g * 2 + u
                        we = ch * CH2 + e
                        we16 = jnp.full((L,), we, jnp.int32)
                        w0s = plsc.load_gather(w0_v, [we16])
                        w1s = plsc.load_gather(w1_v, [we16])
                        for k in range(NHID // L):
                            rbuf[e, pl.ds(k * L, L)] = (
                                rbuf[e, pl.ds(k * L, L)] * w0s)
                            rbuf[e, pl.ds(NHID + k * L, L)] = (
                                rbuf[e, pl.ds(NHID + k * L, L)] * w1s)
                    return 0
                lax.fori_loop(0, CH2 // 2, _scale, 0)

                pltpu.async_copy(
                    rbuf, acc_num.at[di_v.at[ch]], ssem, add=True)
            return 0

        lax.fori_loop(0, PAIRS, _pair, 0)
        return 0

    lax.fori_loop(0, NB2, _batch, 0)

    pltpu.make_async_copy(rows_a, acc_num.at[di_v.at[BCH2 - 2]], sem_sa).wait()
    pltpu.make_async_copy(rows_b, acc_num.at[di_v.at[BCH2 - 1]], sem_sb).wait()

    eoff = t * EPT + NB2 * BCH2 * CH2
    pltpu.sync_copy(src_hbm.at[pl.ds(eoff, CH2)], gi_v.at[pl.ds(0, CH2)])
    pltpu.sync_copy(dst2_hbm.at[pl.ds(eoff // CH2, 1)], di_v.at[pl.ds(0, 1)])
    pltpu.sync_copy(w0_hbm.at[pl.ds(c * E + eoff, CH2)], w0_v.at[pl.ds(0, CH2)])
    pltpu.sync_copy(w1_hbm.at[pl.ds(c * E + eoff, CH2)], w1_v.at[pl.ds(0, CH2)])

    def _ebias(k, _):
        gi_v[pl.ds(k * L, L)] = gi_v[pl.ds(k * L, L)] + cbias
        return 0
    lax.fori_loop(0, CH2 // L, _ebias, 0)
    pltpu.async_copy(wh_hbm.at[gi_v.at[pl.ds(0, CH2)]], rows_a, sem_ga).wait()

    def _escale(g, _):
        for u in range(2):
            e = g * 2 + u
            we16 = jnp.full((L,), e, jnp.int32)
            w0s = plsc.load_gather(w0_v, [we16])
            w1s = plsc.load_gather(w1_v, [we16])
            for k in range(NHID // L):
                rows_a[e, pl.ds(k * L, L)] = rows_a[e, pl.ds(k * L, L)] * w0s
                rows_a[e, pl.ds(NHID + k * L, L)] = (
                    rows_a[e, pl.ds(NHID + k * L, L)] * w1s)
        return 0
    lax.fori_loop(0, CH2 // 2, _escale, 0)
    pltpu.sync_copy(rows_a, acc_num.at[di_v.at[0]], add=True)

    plsc.subcore_barrier()

    def _ochunk(q, _):
        g = q * NT + t
        @pl.when(g < NWC)
        def _():
            roff = g * WCH
            pltpu.sync_copy(acc_num.at[pl.ds(roff, WCH)],
                            out_hbm.at[pl.ds(c * N + roff, WCH)])
        return 0
    lax.fori_loop(0, WPT, _ochunk, 0)


def kernel(x, edge_index, W, a):
    src = edge_index[0]
    dst = edge_index[1]
    Wc = W.transpose(1, 0, 2).reshape(NFEAT, NHEADS * NHID)
    W2 = Wc.reshape(NFEAT, NC, HALF).transpose(1, 0, 2)
    u = jnp.einsum("hfk,hk->hf", W, a[:, :NHID])
    v = jnp.einsum("hfk,hk->hf", W, a[:, NHID:])
    wa = jnp.stack([
        jnp.stack([u[0], u[1], v[0], v[1]], axis=1),
        jnp.stack([u[2], u[3], v[2], v[3]], axis=1),
    ])

    s2 = _tc_s(x, wa)
    w0, w1, den = _sc_weights(
        src.reshape(E // CHUNK, CHUNK), dst.reshape(E // CHUNK, CHUNK), s2)
    wh2 = _tc_wh(x, W2)
    num2 = _sc_scatter(src, dst.reshape(E // CH2, CH2), wh2,
                       w0.reshape(NC * E), w1.reshape(NC * E))
    return _tc_norm(num2, den)
